# Initial kernel scaffold; baseline (speedup 1.0000x reference)
import functools

import jax
import jax.numpy as jnp
from jax import lax
from jax.experimental import pallas as pl
from jax.experimental.pallas import tpu as pltpu
from jax.experimental.pallas import tpu_sc as plsc

_NC = 2
_NS = 16
_L = 16
_NW = _NC * _NS


def _wid():
    return lax.axis_index("s") * _NC + lax.axis_index("c")


def _make_sort_kernel(M, E):
    chunk = M // _NW
    n_vec_total = M // _L
    n_vec_chunk = chunk // _L
    mesh = plsc.VectorSubcoreMesh(core_axis_name="c", subcore_axis_name="s")

    onehots = [
        jnp.where(lax.iota(jnp.int32, _L) == v, jnp.int32(1), jnp.int32(0))
        for v in range(E)
    ]

    @functools.partial(
        pl.kernel,
        out_type=(
            jax.ShapeDtypeStruct((M,), jnp.int32),
            jax.ShapeDtypeStruct((M,), jnp.float32),
        ),
        mesh=mesh,
        scratch_types=[
            pltpu.VMEM((M,), jnp.int32),
            pltpu.VMEM((M,), jnp.float32),
            pltpu.VMEM((chunk,), jnp.int32),
            pltpu.VMEM((chunk,), jnp.float32),
            pltpu.VMEM((2 * _L,), jnp.int32),
        ],
    )
    def sort_kernel(e_hbm, p_hbm, pos_hbm, s_hbm, e_v, p_v, pos_v, s_v, acc_v):
        w = _wid()
        pltpu.sync_copy(e_hbm, e_v)
        pltpu.sync_copy(p_hbm, p_v)

        acc_v[pl.ds(0, _L)] = jnp.zeros((_L,), jnp.int32)
        acc_v[pl.ds(_L, _L)] = jnp.zeros((_L,), jnp.int32)
        first_own = w * n_vec_chunk

        @pl.loop(0, n_vec_total)
        def _(t):
            ev = e_v[pl.ds(t * _L, _L)]
            is_before = jnp.where(t < first_own, jnp.int32(1), jnp.int32(0))
            tot = acc_v[pl.ds(0, _L)]
            bef = acc_v[pl.ds(_L, _L)]
            for v in range(E):
                cnt = plsc.all_reduce_population_count(ev == v)
                tot = tot + cnt * onehots[v]
                bef = bef + (cnt * is_before) * onehots[v]
            acc_v[pl.ds(0, _L)] = tot
            acc_v[pl.ds(_L, _L)] = bef

        total = acc_v[pl.ds(0, _L)]
        before = acc_v[pl.ds(_L, _L)]
        start0 = (plsc.cumsum(total) - total) + before

        def body(t2, start):
            ev = e_v[pl.ds((first_own + t2) * _L, _L)]
            pos_vec = jnp.zeros((_L,), jnp.int32)
            for v in range(E):
                m = ev == v
                mi = jnp.where(m, jnp.int32(1), jnp.int32(0))
                incl = plsc.cumsum(mi)
                base_v = jnp.sum(start * onehots[v])
                pos_vec = jnp.where(m, base_v + incl - 1, pos_vec)
                cnt = plsc.all_reduce_population_count(m)
                start = start + cnt * onehots[v]
            pos_v[pl.ds(t2 * _L, _L)] = pos_vec
            s_v[pl.ds(t2 * _L, _L)] = plsc.load_gather(p_v, [pos_vec])
            return start

        lax.fori_loop(0, n_vec_chunk, body, start0)

        pltpu.sync_copy(pos_v, pos_hbm.at[pl.ds(w * chunk, chunk)])
        pltpu.sync_copy(s_v, s_hbm.at[pl.ds(w * chunk, chunk)])

    return sort_kernel


def _make_scatter_kernel(M, H):
    chunk = M // _NW
    cb = 16
    n_cb = chunk // cb
    mesh = plsc.VectorSubcoreMesh(core_axis_name="c", subcore_axis_name="s")

    @functools.partial(
        pl.kernel,
        out_type=jax.ShapeDtypeStruct((M, H), jnp.float32),
        mesh=mesh,
        scratch_types=[
            pltpu.VMEM((chunk,), jnp.int32),
            pltpu.VMEM((cb, H), jnp.float32),
            pltpu.VMEM((cb, H), jnp.float32),
            pltpu.VMEM((cb,), jnp.int32),
            pltpu.VMEM((cb,), jnp.int32),
            pltpu.SemaphoreType.DMA,
            pltpu.SemaphoreType.DMA,
            pltpu.SemaphoreType.DMA,
            pltpu.SemaphoreType.DMA,
        ],
    )
    def scatter_kernel(z_hbm, pos_hbm, out_hbm, pos_v, buf0, buf1,
                       idx0, idx1, ls0, ls1, ss0, ss1):
        w = _wid()
        row0 = w * chunk
        pltpu.sync_copy(pos_hbm.at[pl.ds(row0, chunk)], pos_v)

        bufs = (buf0, buf1)
        idxs = (idx0, idx1)
        lsems = (ls0, ls1)
        ssems = (ss0, ss1)

        loads = [None, None]
        for c in range(min(2, n_cb)):
            loads[c] = pltpu.async_copy(
                z_hbm.at[pl.ds(row0 + c * cb, cb)], bufs[c], lsems[c])
        for c in range(n_cb):
            b = c & 1
            loads[b].wait()
            idxs[b][...] = pos_v[pl.ds(c * cb, cb)]
            store = pltpu.async_copy(bufs[b], out_hbm.at[idxs[b]], ssems[b])
            store.wait()
            nxt = c + 2
            if nxt < n_cb:
                loads[b] = pltpu.async_copy(
                    z_hbm.at[pl.ds(row0 + nxt * cb, cb)], bufs[b], lsems[b])

    return scatter_kernel


def _mm_body(x_ref, w_ref, be_ref, s_ref, o_ref):
    xb = x_ref[...].astype(jnp.bfloat16)
    acc = jnp.dot(xb, w_ref[...], preferred_element_type=jnp.float32)
    o_ref[...] = (acc + be_ref[...]) * s_ref[...]


def _expert_matmul(hs, We_bf, be, s, bm=512):
    M, H = hs.shape
    return pl.pallas_call(
        _mm_body,
        grid=(M // bm,),
        in_specs=[
            pl.BlockSpec((bm, H), lambda i: (i, 0)),
            pl.BlockSpec((H, H), lambda i: (0, 0)),
            pl.BlockSpec((1, H), lambda i: (0, 0)),
            pl.BlockSpec((bm, 1), lambda i: (i, 0)),
        ],
        out_specs=pl.BlockSpec((bm, H), lambda i: (i, 0)),
        out_shape=jax.ShapeDtypeStruct((M, H), jnp.float32),
    )(hs, We_bf, be.reshape(1, H), s.reshape(M, 1))


def kernel(x, Wg, bg, We, be):
    B, S, H = x.shape
    E = Wg.shape[1]
    M = B * S
    hs = x.reshape(M, H)

    router_logits = hs @ Wg + bg
    normalized_logits = jax.nn.softmax(router_logits, axis=1)
    best = jnp.argmax(normalized_logits, axis=1)
    p = jnp.take_along_axis(normalized_logits, best[:, None], axis=1)[:, 0]

    e = best.astype(jnp.int32)
    pos, s = _make_sort_kernel(M, E)(e, p)
    z = _expert_matmul(hs, We.astype(jnp.bfloat16), be, s)
    out = _make_scatter_kernel(M, H)(z, pos)
    return out.reshape(B, S, H)


# speedup vs baseline: 1.9394x; 1.9394x over previous
import dataclasses
import functools

import jax
import jax.numpy as jnp
from jax import lax
from jax.experimental import pallas as pl
from jax.experimental.pallas import tpu as pltpu
from jax.experimental.pallas import tpu_sc as plsc

_NC = 2
_NS = 16
_L = 16
_NW = _NC * _NS


def _wid():
    return lax.axis_index("s") * _NC + lax.axis_index("c")


def _sc_compiler_params():
    cp = pltpu.CompilerParams()
    if "needs_layout_passes" in pltpu.CompilerParams.__dataclass_fields__:
        cp = dataclasses.replace(cp, needs_layout_passes=False)
    return cp


def _make_sort_kernel(M, E):
    chunk = M // _NW
    n_vec_total = M // _L
    n_vec_chunk = chunk // _L
    mesh = plsc.VectorSubcoreMesh(core_axis_name="c", subcore_axis_name="s")

    @functools.partial(
        pl.kernel,
        out_type=(
            jax.ShapeDtypeStruct((M,), jnp.int32),
            jax.ShapeDtypeStruct((M,), jnp.float32),
        ),
        mesh=mesh,
        scratch_types=[
            pltpu.VMEM((M,), jnp.int32),
            pltpu.VMEM((M,), jnp.float32),
            pltpu.VMEM((chunk,), jnp.int32),
            pltpu.VMEM((chunk,), jnp.float32),
            pltpu.VMEM((2 * _L,), jnp.int32),
        ],
        compiler_params=_sc_compiler_params(),
    )
    def sort_kernel(e_hbm, p_hbm, pos_hbm, s_hbm, e_v, p_v, pos_v, s_v, acc_v):
        w = _wid()
        lanes = lax.iota(jnp.int32, _L)
        onehots = [
            jnp.where(lanes == v, jnp.int32(1), jnp.int32(0)) for v in range(E)
        ]
        pltpu.sync_copy(e_hbm, e_v)
        pltpu.sync_copy(p_hbm, p_v)

        acc_v[pl.ds(0, _L)] = jnp.zeros((_L,), jnp.int32)
        acc_v[pl.ds(_L, _L)] = jnp.zeros((_L,), jnp.int32)
        first_own = w * n_vec_chunk

        @pl.loop(0, n_vec_total)
        def _(t):
            ev = e_v[pl.ds(t * _L, _L)]
            is_before = jnp.where(t < first_own, jnp.int32(1), jnp.int32(0))
            tot = acc_v[pl.ds(0, _L)]
            bef = acc_v[pl.ds(_L, _L)]
            for v in range(E):
                cnt = plsc.all_reduce_population_count(ev == v)
                tot = tot + cnt * onehots[v]
                bef = bef + (cnt * is_before) * onehots[v]
            acc_v[pl.ds(0, _L)] = tot
            acc_v[pl.ds(_L, _L)] = bef

        total = acc_v[pl.ds(0, _L)]
        before = acc_v[pl.ds(_L, _L)]
        start0 = (plsc.cumsum(total) - total) + before

        def body(t2, start):
            ev = e_v[pl.ds((first_own + t2) * _L, _L)]
            pos_vec = jnp.zeros((_L,), jnp.int32)
            for v in range(E):
                m = ev == v
                mi = jnp.where(m, jnp.int32(1), jnp.int32(0))
                incl = plsc.cumsum(mi)
                base_v = jnp.sum(start * onehots[v])
                pos_vec = jnp.where(m, base_v + incl - 1, pos_vec)
                cnt = plsc.all_reduce_population_count(m)
                start = start + cnt * onehots[v]
            pos_v[pl.ds(t2 * _L, _L)] = pos_vec
            s_v[pl.ds(t2 * _L, _L)] = plsc.load_gather(p_v, [pos_vec])
            return start

        lax.fori_loop(0, n_vec_chunk, body, start0)

        pltpu.sync_copy(pos_v, pos_hbm.at[pl.ds(w * chunk, chunk)])
        pltpu.sync_copy(s_v, s_hbm.at[pl.ds(w * chunk, chunk)])

    return sort_kernel


def _make_scatter_kernel(M, H):
    chunk = M // _NW
    cb = 16
    n_cb = chunk // cb
    mesh = plsc.VectorSubcoreMesh(core_axis_name="c", subcore_axis_name="s")

    @functools.partial(
        pl.kernel,
        out_type=jax.ShapeDtypeStruct((M, H), jnp.float32),
        mesh=mesh,
        scratch_types=[
            pltpu.VMEM((chunk,), jnp.int32),
            pltpu.VMEM((cb, H), jnp.float32),
            pltpu.VMEM((cb, H), jnp.float32),
            pltpu.VMEM((cb,), jnp.int32),
            pltpu.VMEM((cb,), jnp.int32),
            pltpu.SemaphoreType.DMA,
            pltpu.SemaphoreType.DMA,
            pltpu.SemaphoreType.DMA,
            pltpu.SemaphoreType.DMA,
        ],
        compiler_params=_sc_compiler_params(),
    )
    def scatter_kernel(z_hbm, pos_hbm, out_hbm, pos_v, buf0, buf1,
                       idx0, idx1, ls0, ls1, ss0, ss1):
        w = _wid()
        row0 = w * chunk
        pltpu.sync_copy(pos_hbm.at[pl.ds(row0, chunk)], pos_v)

        bufs = (buf0, buf1)
        idxs = (idx0, idx1)
        lsems = (ls0, ls1)
        ssems = (ss0, ss1)

        loads = [None, None]
        for c in range(min(2, n_cb)):
            loads[c] = pltpu.async_copy(
                z_hbm.at[pl.ds(row0 + c * cb, cb)], bufs[c], lsems[c])
        for c in range(n_cb):
            b = c & 1
            loads[b].wait()
            idxs[b][...] = pos_v[pl.ds(c * cb, cb)]
            store = pltpu.async_copy(bufs[b], out_hbm.at[idxs[b]], ssems[b])
            store.wait()
            nxt = c + 2
            if nxt < n_cb:
                loads[b] = pltpu.async_copy(
                    z_hbm.at[pl.ds(row0 + nxt * cb, cb)], bufs[b], lsems[b])

    return scatter_kernel


def _mm_body(x_ref, w_ref, be_ref, s_ref, o_ref):
    xb = x_ref[...].astype(jnp.bfloat16)
    acc = jnp.dot(xb, w_ref[...], preferred_element_type=jnp.float32)
    o_ref[...] = (acc + be_ref[...]) * s_ref[...]


def _expert_matmul(hs, We_bf, be, s, bm=512):
    M, H = hs.shape
    return pl.pallas_call(
        _mm_body,
        grid=(M // bm,),
        in_specs=[
            pl.BlockSpec((bm, H), lambda i: (i, 0)),
            pl.BlockSpec((H, H), lambda i: (0, 0)),
            pl.BlockSpec((1, H), lambda i: (0, 0)),
            pl.BlockSpec((bm, 1), lambda i: (i, 0)),
        ],
        out_specs=pl.BlockSpec((bm, H), lambda i: (i, 0)),
        out_shape=jax.ShapeDtypeStruct((M, H), jnp.float32),
    )(hs, We_bf, be.reshape(1, H), s.reshape(M, 1))


def kernel(x, Wg, bg, We, be):
    B, S, H = x.shape
    E = Wg.shape[1]
    M = B * S
    hs = x.reshape(M, H)

    router_logits = hs @ Wg + bg
    normalized_logits = jax.nn.softmax(router_logits, axis=1)
    best = jnp.argmax(normalized_logits, axis=1)
    p = jnp.take_along_axis(normalized_logits, best[:, None], axis=1)[:, 0]

    e = best.astype(jnp.int32)
    pos, s = _make_sort_kernel(M, E)(e, p)
    z = _expert_matmul(hs, We.astype(jnp.bfloat16), be, s)
    out = _make_scatter_kernel(M, H)(z, pos)
    return out.reshape(B, S, H)
